# Initial kernel scaffold; baseline (speedup 1.0000x reference)
import jax
import jax.numpy as jnp
from jax.experimental import pallas as pl


def kernel(x, top_scores, selected_experts_indices, w1, w2, w3):
    raise NotImplementedError("write your pallas kernel here")


# speedup vs baseline: 3.2212x; 3.2212x over previous
import functools

import jax
import jax.numpy as jnp
from jax.experimental import pallas as pl
from jax.experimental.pallas import tpu as pltpu

E = 16
DIM = 1024
HID = 512
N = 4096
K = 2
NK = N * K
T = 256
NT = NK // T
G = NT + E - 1


def _swiglu_body(tt_ref, ee_ref, st_ref, en_ref,
                 x_ref, sc_ref, w1_ref, w3_ref, w2_ref, out_ref):
    g = pl.program_id(0)
    xb = x_ref[...]
    w1e = w1_ref[0]
    w3e = w3_ref[0]
    w2e = w2_ref[0]
    a = jax.lax.dot_general(xb, w1e, (((1,), (1,)), ((), ())),
                            preferred_element_type=jnp.float32)
    b = jax.lax.dot_general(xb, w3e, (((1,), (1,)), ((), ())),
                            preferred_element_type=jnp.float32)
    h = (a * jax.nn.sigmoid(a)) * b
    o = jax.lax.dot_general(h, w2e, (((1,), (1,)), ((), ())),
                            preferred_element_type=jnp.float32)
    o = o * sc_ref[...]
    rows = jax.lax.broadcasted_iota(jnp.int32, (T, 1), 0)
    mask = (rows >= st_ref[g]) & (rows < en_ref[g])
    out_ref[...] = jnp.where(mask, o, out_ref[...])


def _grouped_swiglu(rx, ss, w1, w3, w2, tt, ee, st, en):
    grid_spec = pltpu.PrefetchScalarGridSpec(
        num_scalar_prefetch=4,
        grid=(G,),
        in_specs=[
            pl.BlockSpec((T, DIM), lambda g, tt, ee, st, en: (tt[g], 0)),
            pl.BlockSpec((T, 1), lambda g, tt, ee, st, en: (tt[g], 0)),
            pl.BlockSpec((1, HID, DIM), lambda g, tt, ee, st, en: (ee[g], 0, 0)),
            pl.BlockSpec((1, HID, DIM), lambda g, tt, ee, st, en: (ee[g], 0, 0)),
            pl.BlockSpec((1, DIM, HID), lambda g, tt, ee, st, en: (ee[g], 0, 0)),
        ],
        out_specs=pl.BlockSpec((T, DIM), lambda g, tt, ee, st, en: (tt[g], 0)),
    )
    return pl.pallas_call(
        _swiglu_body,
        grid_spec=grid_spec,
        out_shape=jax.ShapeDtypeStruct((NK, DIM), jnp.float32),
    )(tt, ee, st, en, rx, ss, w1, w3, w2)


def _group_metadata(flat_exp):
    sizes = jnp.bincount(flat_exp, length=E)
    off = jnp.concatenate([jnp.zeros((1,), jnp.int32),
                           jnp.cumsum(sizes).astype(jnp.int32)])
    first_tile = off[:E] // T
    last_tile = (off[1:] - 1) // T
    tiles_e = jnp.where(sizes > 0, last_tile - first_tile + 1, 0).astype(jnp.int32)
    cum = jnp.cumsum(tiles_e)
    total = cum[-1]
    gids = jnp.arange(G, dtype=jnp.int32)
    e_of_g = jnp.searchsorted(cum, gids, side="right").astype(jnp.int32)
    valid = gids < total
    e_cl = jnp.minimum(e_of_g, E - 1)
    local = gids - (cum[e_cl] - tiles_e[e_cl])
    t_of_g = first_tile[e_cl] + local
    t_of_g = jnp.where(valid, t_of_g, NT - 1).astype(jnp.int32)
    ee = jnp.where(valid, e_cl, E - 1).astype(jnp.int32)
    st = jnp.where(valid, jnp.clip(off[e_cl] - t_of_g * T, 0, T), 0).astype(jnp.int32)
    en = jnp.where(valid, jnp.clip(off[e_cl + 1] - t_of_g * T, 0, T), 0).astype(jnp.int32)
    return t_of_g, ee, st, en


def kernel(x, top_scores, selected_experts_indices, w1, w2, w3):
    flat_exp = selected_experts_indices.reshape(-1)
    sort_idx = jnp.argsort(flat_exp, stable=True).astype(jnp.int32)
    tok_sorted = sort_idx // K
    inv_perm = jnp.zeros((NK,), jnp.int32).at[sort_idx].set(
        jnp.arange(NK, dtype=jnp.int32))
    tt, ee, st, en = _group_metadata(flat_exp)

    rx = x[tok_sorted]
    ss = top_scores.reshape(-1)[sort_idx].reshape(NK, 1)

    ro = _grouped_swiglu(rx, ss, w1, w3, w2, tt, ee, st, en)

    g = ro[inv_perm]
    out = g.reshape(N, K, DIM).sum(axis=1)
    return out
